# baseline (speedup 1.0000x reference)
import jax
import jax.numpy as jnp
from jax import lax
from jax.experimental import pallas as pl
from jax.experimental.pallas import tpu as pltpu
from jax.experimental.pallas import tpu_sc as plsc

B = 16384
D = 64
NC, NS, L = 2, 16, 16
NW = NC * NS
C = B // NW
NK = C // L
NJ = D // L
DB = 8
EPS = 1e-5


def _sc_body(xt_hbm, c1_hbm, c4_hbm, c5_hbm, emb_hbm, wt_hbm, bm_hbm,
             lnw_hbm, lnb_hbm, out_hbm,
             x_v, c1_v, c4_v, c5_v, al_v, be_v, ga_v, dl_v,
             emb_v, wt_v, bm_v, lnw_v, lnb_v, sin):
    wid = lax.axis_index("s") * NC + lax.axis_index("c")
    base = wid * C

    hin = pltpu.async_copy(xt_hbm.at[:, pl.ds(base, C)], x_v, sin)

    pltpu.sync_copy(c1_hbm.at[pl.ds(base, C)], c1_v)
    pltpu.sync_copy(c4_hbm.at[pl.ds(base, C)], c4_v)
    pltpu.sync_copy(c5_hbm.at[pl.ds(base, C)], c5_v)
    pltpu.sync_copy(emb_hbm, emb_v)
    pltpu.sync_copy(wt_hbm, wt_v)
    pltpu.sync_copy(bm_hbm, bm_v)
    pltpu.sync_copy(lnw_hbm, lnw_v)
    pltpu.sync_copy(lnb_hbm, lnb_v)

    w0 = [wt_v[0, pl.ds(j * L, L)] for j in range(NJ)]
    w1 = [wt_v[1, pl.ds(j * L, L)] for j in range(NJ)]
    bm = [bm_v[pl.ds(j * L, L)] for j in range(NJ)]
    lnw = [lnw_v[pl.ds(j * L, L)] for j in range(NJ)]
    lnb = [lnb_v[pl.ds(j * L, L)] for j in range(NJ)]
    e0 = [emb_v[0, pl.ds(j * L, L)] for j in range(NJ)]
    e1 = [emb_v[1, pl.ds(j * L, L)] for j in range(NJ)]

    def vsum(chunks):
        acc = chunks[0]
        for ch in chunks[1:]:
            acc = acc + ch
        return jnp.sum(acc)

    inv_n = 1.0 / D
    mw0 = vsum(w0) * inv_n
    mw1 = vsum(w1) * inv_n
    mb = vsum(bm) * inv_n
    A = vsum([w0[j] * w0[j] for j in range(NJ)]) * inv_n - mw0 * mw0
    Bq = vsum([w1[j] * w1[j] for j in range(NJ)]) * inv_n - mw1 * mw1
    C2 = 2.0 * (vsum([w0[j] * w1[j] for j in range(NJ)]) * inv_n - mw0 * mw1)
    D2 = 2.0 * (vsum([w0[j] * bm[j] for j in range(NJ)]) * inv_n - mw0 * mb)
    E2 = 2.0 * (vsum([w1[j] * bm[j] for j in range(NJ)]) * inv_n - mw1 * mb)
    F = vsum([bm[j] * bm[j] for j in range(NJ)]) * inv_n - mb * mb + EPS

    U = [(w0[j] - mw0) * lnw[j] for j in range(NJ)]
    V = [(w1[j] - mw1) * lnw[j] for j in range(NJ)]
    Tw = [(bm[j] - mb) * lnw[j] for j in range(NJ)]
    T2 = [e0[j] + lnb[j] for j in range(NJ)]
    Dl = [e1[j] - e0[j] for j in range(NJ)]

    def scal(k, _):
        sl = pl.ds(k * L, L)
        c4g = c4_v[sl]
        c5g = c5_v[sl]
        var = (A * c4g * c4g + Bq * c5g * c5g + C2 * c4g * c5g
               + D2 * c4g + E2 * c5g + F)
        i = lax.bitcast_convert_type(var, jnp.int32)
        i = 0x5F3759DF - lax.shift_right_arithmetic(i, 1)
        y = lax.bitcast_convert_type(i, jnp.float32)
        for _ in range(3):
            y = y * (1.5 - 0.5 * var * y * y)
        al_v[sl] = y * c4g
        be_v[sl] = y * c5g
        ga_v[sl] = y
        dl_v[sl] = c1_v[sl].astype(jnp.float32)
        return 0

    lax.fori_loop(0, NK, scal, 0)

    hin.wait()

    for db in range(D // DB):
        ds_ = [db * DB + i for i in range(DB)]
        sc5 = [(U[d // L][d % L], V[d // L][d % L], Tw[d // L][d % L],
                T2[d // L][d % L], Dl[d // L][d % L]) for d in ds_]

        def col(k, _, ds_=ds_, sc5=sc5):
            sl = pl.ds(k * L, L)
            al = al_v[sl]
            be = be_v[sl]
            ga = ga_v[sl]
            dl = dl_v[sl]
            for d, (ud, vd, twd, t2d, dld) in zip(ds_, sc5):
                x_v[d, sl] = (x_v[d, sl] + t2d + dld * dl
                              + ud * al + vd * be + twd * ga)
            return 0

        lax.fori_loop(0, NK, col, 0, unroll=2)

    pltpu.sync_copy(x_v, out_hbm.at[:, pl.ds(base, C)])


def kernel(cond0, cond1, cond4, cond5, emb_table, W_meta, b_meta, ln_w, ln_b):
    mesh = plsc.VectorSubcoreMesh(core_axis_name="c", subcore_axis_name="s")
    f = pl.kernel(
        _sc_body,
        out_type=jax.ShapeDtypeStruct((D, B), jnp.float32),
        mesh=mesh,
        compiler_params=pltpu.CompilerParams(needs_layout_passes=False),
        scratch_types=[
            pltpu.VMEM((D, C), jnp.float32),
            pltpu.VMEM((C,), jnp.int32),
            pltpu.VMEM((C,), jnp.float32),
            pltpu.VMEM((C,), jnp.float32),
            pltpu.VMEM((C,), jnp.float32),
            pltpu.VMEM((C,), jnp.float32),
            pltpu.VMEM((C,), jnp.float32),
            pltpu.VMEM((C,), jnp.float32),
            pltpu.VMEM((2, D), jnp.float32),
            pltpu.VMEM((2, D), jnp.float32),
            pltpu.VMEM((D,), jnp.float32),
            pltpu.VMEM((D,), jnp.float32),
            pltpu.VMEM((D,), jnp.float32),
            pltpu.SemaphoreType.DMA,
        ],
    )
    out_t = f(cond0.T, cond1, cond4.reshape(B), cond5.reshape(B),
              emb_table, W_meta.T, b_meta, ln_w, ln_b)
    return out_t.T


# speedup vs baseline: 1.6271x; 1.6271x over previous
import jax
import jax.numpy as jnp
from jax import lax
from jax.experimental import pallas as pl
from jax.experimental.pallas import tpu as pltpu
from jax.experimental.pallas import tpu_sc as plsc

B = 16384
D = 64
NC, NS, L = 2, 16, 16
NW = NC * NS
C = B // NW
NK = C // L
NJ = D // L
DB = 8
EPS = 1e-5


def _prep_body(emb_ref, wt_ref, bm_ref, lnw_ref, lnb_ref, out_ref):
    w0 = wt_ref[0:1, :]
    w1 = wt_ref[1:2, :]
    bm = bm_ref[...]
    lnw = lnw_ref[...]
    lnb = lnb_ref[...]
    e0 = emb_ref[0:1, :]
    e1 = emb_ref[1:2, :]

    u = w0 - jnp.mean(w0)
    v = w1 - jnp.mean(w1)
    t = bm - jnp.mean(bm)

    A = jnp.mean(u * u)
    Bq = jnp.mean(v * v)
    C2 = 2.0 * jnp.mean(u * v)
    D2 = 2.0 * jnp.mean(u * t)
    E2 = 2.0 * jnp.mean(v * t)
    F = jnp.mean(t * t) + EPS

    lane = lax.broadcasted_iota(jnp.int32, (1, D), 1)
    srow = (jnp.where(lane == 0, A, 0.0) + jnp.where(lane == 1, Bq, 0.0)
            + jnp.where(lane == 2, C2, 0.0) + jnp.where(lane == 3, D2, 0.0)
            + jnp.where(lane == 4, E2, 0.0) + jnp.where(lane == 5, F, 0.0))

    out_ref[...] = jnp.concatenate(
        [u * lnw, v * lnw, t * lnw, e0 + lnb, e1 - e0, srow,
         jnp.zeros((2, D), jnp.float32)], axis=0)


def _sc_body(xt_hbm, c1_hbm, c4_hbm, c5_hbm, wp_hbm, out_hbm,
             x_v, c1_v, c4_v, c5_v, al_v, be_v, ga_v, dl_v, wp_v, sin):
    wid = lax.axis_index("s") * NC + lax.axis_index("c")
    base = wid * C

    hin = pltpu.async_copy(xt_hbm.at[:, pl.ds(base, C)], x_v, sin)

    pltpu.sync_copy(c1_hbm.at[pl.ds(base, C)], c1_v)
    pltpu.sync_copy(c4_hbm.at[pl.ds(base, C)], c4_v)
    pltpu.sync_copy(c5_hbm.at[pl.ds(base, C)], c5_v)
    pltpu.sync_copy(wp_hbm, wp_v)

    U = [wp_v[0, pl.ds(j * L, L)] for j in range(NJ)]
    V = [wp_v[1, pl.ds(j * L, L)] for j in range(NJ)]
    Tw = [wp_v[2, pl.ds(j * L, L)] for j in range(NJ)]
    T2 = [wp_v[3, pl.ds(j * L, L)] for j in range(NJ)]
    Dl = [wp_v[4, pl.ds(j * L, L)] for j in range(NJ)]
    s = wp_v[5, pl.ds(0, L)]
    A, Bq, C2, D2, E2, F = s[0], s[1], s[2], s[3], s[4], s[5]

    def scal(k, _):
        sl = pl.ds(k * L, L)
        c4g = c4_v[sl]
        c5g = c5_v[sl]
        var = (A * c4g * c4g + Bq * c5g * c5g + C2 * c4g * c5g
               + D2 * c4g + E2 * c5g + F)
        i = lax.bitcast_convert_type(var, jnp.int32)
        i = 0x5F3759DF - lax.shift_right_arithmetic(i, 1)
        y = lax.bitcast_convert_type(i, jnp.float32)
        for _ in range(3):
            y = y * (1.5 - 0.5 * var * y * y)
        al_v[sl] = y * c4g
        be_v[sl] = y * c5g
        ga_v[sl] = y
        dl_v[sl] = c1_v[sl].astype(jnp.float32)
        return 0

    lax.fori_loop(0, NK, scal, 0)

    hin.wait()

    for db in range(D // DB):
        ds_ = [db * DB + i for i in range(DB)]
        sc5 = [(U[d // L][d % L], V[d // L][d % L], Tw[d // L][d % L],
                T2[d // L][d % L], Dl[d // L][d % L]) for d in ds_]

        def col(k, _, ds_=ds_, sc5=sc5):
            sl = pl.ds(k * L, L)
            al = al_v[sl]
            be = be_v[sl]
            ga = ga_v[sl]
            dl = dl_v[sl]
            for d, (ud, vd, twd, t2d, dld) in zip(ds_, sc5):
                x_v[d, sl] = (x_v[d, sl] + t2d + dld * dl
                              + ud * al + vd * be + twd * ga)
            return 0

        lax.fori_loop(0, NK, col, 0)

    pltpu.sync_copy(x_v, out_hbm.at[:, pl.ds(base, C)])


def kernel(cond0, cond1, cond4, cond5, emb_table, W_meta, b_meta, ln_w, ln_b):
    wpack = pl.pallas_call(
        _prep_body,
        out_shape=jax.ShapeDtypeStruct((8, D), jnp.float32),
    )(emb_table, W_meta.T, b_meta.reshape(1, D), ln_w.reshape(1, D),
      ln_b.reshape(1, D))

    mesh = plsc.VectorSubcoreMesh(core_axis_name="c", subcore_axis_name="s")
    f = pl.kernel(
        _sc_body,
        out_type=jax.ShapeDtypeStruct((D, B), jnp.float32),
        mesh=mesh,
        scratch_types=[
            pltpu.VMEM((D, C), jnp.float32),
            pltpu.VMEM((C,), jnp.int32),
            pltpu.VMEM((C,), jnp.float32),
            pltpu.VMEM((C,), jnp.float32),
            pltpu.VMEM((C,), jnp.float32),
            pltpu.VMEM((C,), jnp.float32),
            pltpu.VMEM((C,), jnp.float32),
            pltpu.VMEM((C,), jnp.float32),
            pltpu.VMEM((8, D), jnp.float32),
            pltpu.SemaphoreType.DMA,
        ],
    )
    out_t = f(cond0.T, cond1, cond4.reshape(B), cond5.reshape(B), wpack)
    return out_t.T


# speedup vs baseline: 1.6602x; 1.0204x over previous
import jax
import jax.numpy as jnp
from jax import lax
from jax.experimental import pallas as pl
from jax.experimental.pallas import tpu as pltpu
from jax.experimental.pallas import tpu_sc as plsc

B = 16384
D = 64
NC, NS, L = 2, 16, 16
NW = NC * NS
C = B // NW
NK = C // L
NJ = D // L
DB = 8
EPS = 1e-5


def _prep_body(emb_ref, wt_ref, bm_ref, lnw_ref, lnb_ref, out_ref):
    w0 = wt_ref[0:1, :]
    w1 = wt_ref[1:2, :]
    bm = bm_ref[...]
    lnw = lnw_ref[...]
    lnb = lnb_ref[...]
    e0 = emb_ref[0:1, :]
    e1 = emb_ref[1:2, :]

    u = w0 - jnp.mean(w0)
    v = w1 - jnp.mean(w1)
    t = bm - jnp.mean(bm)

    A = jnp.mean(u * u)
    Bq = jnp.mean(v * v)
    C2 = 2.0 * jnp.mean(u * v)
    D2 = 2.0 * jnp.mean(u * t)
    E2 = 2.0 * jnp.mean(v * t)
    F = jnp.mean(t * t) + EPS

    lane = lax.broadcasted_iota(jnp.int32, (1, D), 1)
    srow = (jnp.where(lane == 0, A, 0.0) + jnp.where(lane == 1, Bq, 0.0)
            + jnp.where(lane == 2, C2, 0.0) + jnp.where(lane == 3, D2, 0.0)
            + jnp.where(lane == 4, E2, 0.0) + jnp.where(lane == 5, F, 0.0))

    out_ref[...] = jnp.concatenate(
        [u * lnw, v * lnw, t * lnw, e0 + lnb, e1 - e0, srow,
         jnp.zeros((2, D), jnp.float32)], axis=0)


def _sc_body(xt_hbm, c1_hbm, c4_hbm, c5_hbm, wp_hbm, out_hbm,
             x_v, c1_v, c4_v, c5_v, al_v, be_v, ga_v, dl_v, wp_v, *sems):
    wid = lax.axis_index("s") * NC + lax.axis_index("c")
    base = wid * C
    nb = D // DB

    hin = [pltpu.async_copy(
        xt_hbm.at[pl.ds(db * DB, DB), pl.ds(base, C)],
        x_v.at[pl.ds(db * DB, DB), :], sems[db]) for db in range(nb)]

    pltpu.sync_copy(c1_hbm.at[pl.ds(base, C)], c1_v)
    pltpu.sync_copy(c4_hbm.at[pl.ds(base, C)], c4_v)
    pltpu.sync_copy(c5_hbm.at[pl.ds(base, C)], c5_v)
    pltpu.sync_copy(wp_hbm, wp_v)

    U = [wp_v[0, pl.ds(j * L, L)] for j in range(NJ)]
    V = [wp_v[1, pl.ds(j * L, L)] for j in range(NJ)]
    Tw = [wp_v[2, pl.ds(j * L, L)] for j in range(NJ)]
    T2 = [wp_v[3, pl.ds(j * L, L)] for j in range(NJ)]
    Dl = [wp_v[4, pl.ds(j * L, L)] for j in range(NJ)]
    s = wp_v[5, pl.ds(0, L)]
    A, Bq, C2, D2, E2, F = s[0], s[1], s[2], s[3], s[4], s[5]

    def scal(k, _):
        sl = pl.ds(k * L, L)
        c4g = c4_v[sl]
        c5g = c5_v[sl]
        var = (A * c4g * c4g + Bq * c5g * c5g + C2 * c4g * c5g
               + D2 * c4g + E2 * c5g + F)
        i = lax.bitcast_convert_type(var, jnp.int32)
        i = 0x5F3759DF - lax.shift_right_arithmetic(i, 1)
        y = lax.bitcast_convert_type(i, jnp.float32)
        for _ in range(3):
            y = y * (1.5 - 0.5 * var * y * y)
        al_v[sl] = y * c4g
        be_v[sl] = y * c5g
        ga_v[sl] = y
        dl_v[sl] = c1_v[sl].astype(jnp.float32)
        return 0

    lax.fori_loop(0, NK, scal, 0)

    hout = []
    for db in range(D // DB):
        hin[db].wait()
        ds_ = [db * DB + i for i in range(DB)]
        sc5 = [(U[d // L][d % L], V[d // L][d % L], Tw[d // L][d % L],
                T2[d // L][d % L], Dl[d // L][d % L]) for d in ds_]

        def col(k, _, ds_=ds_, sc5=sc5):
            sl = pl.ds(k * L, L)
            al = al_v[sl]
            be = be_v[sl]
            ga = ga_v[sl]
            dl = dl_v[sl]
            for d, (ud, vd, twd, t2d, dld) in zip(ds_, sc5):
                x_v[d, sl] = (x_v[d, sl] + t2d + dld * dl
                              + ud * al + vd * be + twd * ga)
            return 0

        lax.fori_loop(0, NK, col, 0)
        hout.append(pltpu.async_copy(
            x_v.at[pl.ds(db * DB, DB), :],
            out_hbm.at[pl.ds(db * DB, DB), pl.ds(base, C)],
            sems[D // DB + db]))

    for ho in hout:
        ho.wait()


def kernel(cond0, cond1, cond4, cond5, emb_table, W_meta, b_meta, ln_w, ln_b):
    wpack = pl.pallas_call(
        _prep_body,
        out_shape=jax.ShapeDtypeStruct((8, D), jnp.float32),
    )(emb_table, W_meta.T, b_meta.reshape(1, D), ln_w.reshape(1, D),
      ln_b.reshape(1, D))

    mesh = plsc.VectorSubcoreMesh(core_axis_name="c", subcore_axis_name="s")
    f = pl.kernel(
        _sc_body,
        out_type=jax.ShapeDtypeStruct((D, B), jnp.float32),
        mesh=mesh,
        scratch_types=[
            pltpu.VMEM((D, C), jnp.float32),
            pltpu.VMEM((C,), jnp.int32),
            pltpu.VMEM((C,), jnp.float32),
            pltpu.VMEM((C,), jnp.float32),
            pltpu.VMEM((C,), jnp.float32),
            pltpu.VMEM((C,), jnp.float32),
            pltpu.VMEM((C,), jnp.float32),
            pltpu.VMEM((C,), jnp.float32),
            pltpu.VMEM((8, D), jnp.float32),
        ] + [pltpu.SemaphoreType.DMA] * 16,
    )
    out_t = f(cond0.T, cond1, cond4.reshape(B), cond5.reshape(B), wpack)
    return out_t.T
